# Initial kernel scaffold; baseline (speedup 1.0000x reference)
import functools
import jax
import jax.numpy as jnp
from jax import lax
from jax.experimental import pallas as pl
from jax.experimental.pallas import tpu as pltpu

BN_BLK = 512
S = 1024
HI = jnp.float32(3.4e38)


def _top3(dists, iota_s):
    d = dists
    vals, idxs = [], []
    for _ in range(3):
        m = jnp.min(d, axis=0)
        is_min = d == m[None, :]
        ik = jnp.min(jnp.where(is_min, iota_s, S), axis=0)
        vals.append(m)
        idxs.append(ik)
        d = jnp.where(iota_s == ik[None, :], HI, d)
    return vals, idxs


def _p1_kernel(x1_ref, x2_ref, p1_ref, p2_ref, w1_ref, b1_ref, h1_ref,
               st1_ref, acc_ref, *, nj):
    b = pl.program_id(0)
    j = pl.program_id(1)

    x1 = x1_ref[0]
    x2 = x2_ref[0]
    dots = jax.lax.dot_general(x2, x1, (((0,), (0,)), ((), ())),
                               preferred_element_type=jnp.float32,
                               precision=lax.Precision.HIGHEST)
    x1sq = jnp.sum(x1 * x1, axis=0)
    x2sq = jnp.sum(x2 * x2, axis=0)
    dists = x1sq[None, :] + x2sq[:, None] - 2.0 * dots

    iota_s = lax.broadcasted_iota(jnp.int32, dists.shape, 0)
    vals, idxs = _top3(dists, iota_s)

    r0 = 1.0 / (vals[0] + 1e-8)
    r1 = 1.0 / (vals[1] + 1e-8)
    r2 = 1.0 / (vals[2] + 1e-8)
    norm = r0 + r1 + r2
    w0, w1w, w2w = r0 / norm, r1 / norm, r2 / norm

    onehot = (jnp.where(iota_s == idxs[0][None, :], w0[None, :], 0.0)
              + jnp.where(iota_s == idxs[1][None, :], w1w[None, :], 0.0)
              + jnp.where(iota_s == idxs[2][None, :], w2w[None, :], 0.0))

    p2 = p2_ref[0]
    interp = jax.lax.dot_general(p2, onehot, (((1,), (0,)), ((), ())),
                                 preferred_element_type=jnp.float32,
                                 precision=lax.Precision.HIGHEST)

    W1 = w1_ref[...]
    p1 = p1_ref[0]
    h1 = (jax.lax.dot_general(W1[:, :128], p1, (((1,), (0,)), ((), ())),
                              preferred_element_type=jnp.float32,
                              precision=lax.Precision.HIGHEST)
          + jax.lax.dot_general(W1[:, 128:], interp, (((1,), (0,)), ((), ())),
                                preferred_element_type=jnp.float32,
                                precision=lax.Precision.HIGHEST)
          + b1_ref[...])
    h1_ref[0] = h1

    @pl.when(jnp.logical_and(b == 0, j == 0))
    def _init():
        acc_ref[...] = jnp.zeros_like(acc_ref)

    acc_ref[0, :] += jnp.sum(h1, axis=1)
    acc_ref[1, :] += jnp.sum(h1 * h1, axis=1)

    @pl.when(jnp.logical_and(b == pl.num_programs(0) - 1, j == nj - 1))
    def _fin():
        st1_ref[...] = acc_ref[...]


def _p2_kernel(h1_ref, w2_ref, b2_ref, sc1_ref, sh1_ref, h2_ref, st2_ref,
               acc_ref, *, nj):
    b = pl.program_id(0)
    j = pl.program_id(1)
    a1 = jnp.maximum(sc1_ref[...] * h1_ref[0] + sh1_ref[...], 0.0)
    h2 = (jax.lax.dot_general(w2_ref[...], a1, (((1,), (0,)), ((), ())),
                              preferred_element_type=jnp.float32,
                              precision=lax.Precision.HIGHEST)
          + b2_ref[...])
    h2_ref[0] = h2

    @pl.when(jnp.logical_and(b == 0, j == 0))
    def _init():
        acc_ref[...] = jnp.zeros_like(acc_ref)

    acc_ref[0, :] += jnp.sum(h2, axis=1)
    acc_ref[1, :] += jnp.sum(h2 * h2, axis=1)

    @pl.when(jnp.logical_and(b == pl.num_programs(0) - 1, j == nj - 1))
    def _fin():
        st2_ref[...] = acc_ref[...]


def _p3_kernel(h2_ref, sc2_ref, sh2_ref, o_ref):
    o_ref[0] = jnp.maximum(sc2_ref[...] * h2_ref[0] + sh2_ref[...], 0.0)


def _affine(stats, g, be, count):
    mean = stats[0] / count
    var = stats[1] / count - mean * mean
    scale = g / jnp.sqrt(var + 1e-5)
    shift = be - scale * mean
    return scale[:, None], shift[:, None]


def kernel(xyz1, xyz2, points1, points2, W1, b1, g1, be1, W2, b2, g2, be2):
    B, _, N = xyz1.shape
    D2 = points2.shape[1]
    bn = BN_BLK
    nj = N // bn

    h1, st1 = pl.pallas_call(
        functools.partial(_p1_kernel, nj=nj),
        grid=(B, nj),
        in_specs=[
            pl.BlockSpec((1, 3, bn), lambda b, j: (b, 0, j)),
            pl.BlockSpec((1, 3, S), lambda b, j: (b, 0, 0)),
            pl.BlockSpec((1, 128, bn), lambda b, j: (b, 0, j)),
            pl.BlockSpec((1, D2, S), lambda b, j: (b, 0, 0)),
            pl.BlockSpec((256, 384), lambda b, j: (0, 0)),
            pl.BlockSpec((256, 1), lambda b, j: (0, 0)),
        ],
        out_specs=[
            pl.BlockSpec((1, 256, bn), lambda b, j: (b, 0, j)),
            pl.BlockSpec((2, 256), lambda b, j: (0, 0)),
        ],
        out_shape=[
            jax.ShapeDtypeStruct((B, 256, N), jnp.float32),
            jax.ShapeDtypeStruct((2, 256), jnp.float32),
        ],
        scratch_shapes=[pltpu.VMEM((2, 256), jnp.float32)],
    )(xyz1, xyz2, points1, points2, W1, b1[:, None])

    sc1, sh1 = _affine(st1, g1, be1, float(B * N))

    h2, st2 = pl.pallas_call(
        functools.partial(_p2_kernel, nj=nj),
        grid=(B, nj),
        in_specs=[
            pl.BlockSpec((1, 256, bn), lambda b, j: (b, 0, j)),
            pl.BlockSpec((128, 256), lambda b, j: (0, 0)),
            pl.BlockSpec((128, 1), lambda b, j: (0, 0)),
            pl.BlockSpec((256, 1), lambda b, j: (0, 0)),
            pl.BlockSpec((256, 1), lambda b, j: (0, 0)),
        ],
        out_specs=[
            pl.BlockSpec((1, 128, bn), lambda b, j: (b, 0, j)),
            pl.BlockSpec((2, 128), lambda b, j: (0, 0)),
        ],
        out_shape=[
            jax.ShapeDtypeStruct((B, 128, N), jnp.float32),
            jax.ShapeDtypeStruct((2, 128), jnp.float32),
        ],
        scratch_shapes=[pltpu.VMEM((2, 128), jnp.float32)],
    )(h1, W2, b2[:, None], sc1, sh1)

    sc2, sh2 = _affine(st2, g2, be2, float(B * N))

    out = pl.pallas_call(
        _p3_kernel,
        grid=(B, nj),
        in_specs=[
            pl.BlockSpec((1, 128, bn), lambda b, j: (b, 0, j)),
            pl.BlockSpec((128, 1), lambda b, j: (0, 0)),
            pl.BlockSpec((128, 1), lambda b, j: (0, 0)),
        ],
        out_specs=pl.BlockSpec((1, 128, bn), lambda b, j: (b, 0, j)),
        out_shape=jax.ShapeDtypeStruct((B, 128, N), jnp.float32),
    )(h2, sc2, sh2)

    return out


# speedup vs baseline: 16.6222x; 16.6222x over previous
import functools
import jax
import jax.numpy as jnp
from jax import lax
from jax.experimental import pallas as pl
from jax.experimental.pallas import tpu as pltpu

BN_BLK = 512
S = 1024
HI = 3.4e38


def _top3(dists, iota_s):
    d = dists
    vals, idxs = [], []
    for _ in range(3):
        m = jnp.min(d, axis=0)
        is_min = d == m[None, :]
        ik = jnp.min(jnp.where(is_min, iota_s, S), axis=0)
        vals.append(m)
        idxs.append(ik)
        d = jnp.where(iota_s == ik[None, :], HI, d)
    return vals, idxs


def _p1_kernel(x1_ref, x2_ref, p1_ref, p2_ref, w1_ref, b1_ref, h1_ref,
               st1_ref, acc_ref, *, nj):
    b = pl.program_id(0)
    j = pl.program_id(1)

    x1 = x1_ref[0]
    x2 = x2_ref[0]
    dots = jax.lax.dot_general(x2, x1, (((0,), (0,)), ((), ())),
                               preferred_element_type=jnp.float32,
                               precision=lax.Precision.DEFAULT)
    x1sq = jnp.sum(x1 * x1, axis=0)
    x2sq = jnp.sum(x2 * x2, axis=0)
    dists = (-2.0 * dots + x1sq[None, :]) + x2sq[:, None]

    iota_s = lax.broadcasted_iota(jnp.int32, dists.shape, 0)
    vals, idxs = _top3(dists, iota_s)

    r0 = 1.0 / (vals[0] + 1e-8)
    r1 = 1.0 / (vals[1] + 1e-8)
    r2 = 1.0 / (vals[2] + 1e-8)
    norm = r0 + r1 + r2
    w0, w1w, w2w = r0 / norm, r1 / norm, r2 / norm

    onehot = (jnp.where(iota_s == idxs[0][None, :], w0[None, :], 0.0)
              + jnp.where(iota_s == idxs[1][None, :], w1w[None, :], 0.0)
              + jnp.where(iota_s == idxs[2][None, :], w2w[None, :], 0.0))

    p2 = p2_ref[0]
    interp = jax.lax.dot_general(p2, onehot, (((1,), (0,)), ((), ())),
                                 preferred_element_type=jnp.float32,
                                 precision=lax.Precision.HIGHEST)

    W1 = w1_ref[...]
    p1 = p1_ref[0]
    h1 = (jax.lax.dot_general(W1[:, :128], p1, (((1,), (0,)), ((), ())),
                              preferred_element_type=jnp.float32,
                              precision=lax.Precision.HIGHEST)
          + jax.lax.dot_general(W1[:, 128:], interp, (((1,), (0,)), ((), ())),
                                preferred_element_type=jnp.float32,
                                precision=lax.Precision.HIGHEST)
          + b1_ref[...])
    h1_ref[0] = h1

    @pl.when(jnp.logical_and(b == 0, j == 0))
    def _init():
        acc_ref[...] = jnp.zeros_like(acc_ref)

    acc_ref[0, :] += jnp.sum(h1, axis=1)
    acc_ref[1, :] += jnp.sum(h1 * h1, axis=1)

    @pl.when(jnp.logical_and(b == pl.num_programs(0) - 1, j == nj - 1))
    def _fin():
        st1_ref[...] = acc_ref[...]


def _p2_kernel(h1_ref, w2_ref, b2_ref, sc1_ref, sh1_ref, h2_ref, st2_ref,
               acc_ref, *, nj):
    b = pl.program_id(0)
    j = pl.program_id(1)
    a1 = jnp.maximum(sc1_ref[...] * h1_ref[0] + sh1_ref[...], 0.0)
    h2 = (jax.lax.dot_general(w2_ref[...], a1, (((1,), (0,)), ((), ())),
                              preferred_element_type=jnp.float32,
                              precision=lax.Precision.HIGHEST)
          + b2_ref[...])
    h2_ref[0] = h2

    @pl.when(jnp.logical_and(b == 0, j == 0))
    def _init():
        acc_ref[...] = jnp.zeros_like(acc_ref)

    acc_ref[0, :] += jnp.sum(h2, axis=1)
    acc_ref[1, :] += jnp.sum(h2 * h2, axis=1)

    @pl.when(jnp.logical_and(b == pl.num_programs(0) - 1, j == nj - 1))
    def _fin():
        st2_ref[...] = acc_ref[...]


def _p3_kernel(h2_ref, sc2_ref, sh2_ref, o_ref):
    o_ref[0] = jnp.maximum(sc2_ref[...] * h2_ref[0] + sh2_ref[...], 0.0)


def _affine(stats, g, be, count):
    mean = stats[0] / count
    var = stats[1] / count - mean * mean
    scale = g / jnp.sqrt(var + 1e-5)
    shift = be - scale * mean
    return scale[:, None], shift[:, None]


def kernel(xyz1, xyz2, points1, points2, W1, b1, g1, be1, W2, b2, g2, be2):
    B, _, N = xyz1.shape
    D2 = points2.shape[1]
    bn = BN_BLK
    nj = N // bn

    h1, st1 = pl.pallas_call(
        functools.partial(_p1_kernel, nj=nj),
        grid=(B, nj),
        in_specs=[
            pl.BlockSpec((1, 3, bn), lambda b, j: (b, 0, j)),
            pl.BlockSpec((1, 3, S), lambda b, j: (b, 0, 0)),
            pl.BlockSpec((1, 128, bn), lambda b, j: (b, 0, j)),
            pl.BlockSpec((1, D2, S), lambda b, j: (b, 0, 0)),
            pl.BlockSpec((256, 384), lambda b, j: (0, 0)),
            pl.BlockSpec((256, 1), lambda b, j: (0, 0)),
        ],
        out_specs=[
            pl.BlockSpec((1, 256, bn), lambda b, j: (b, 0, j)),
            pl.BlockSpec((2, 256), lambda b, j: (0, 0)),
        ],
        out_shape=[
            jax.ShapeDtypeStruct((B, 256, N), jnp.float32),
            jax.ShapeDtypeStruct((2, 256), jnp.float32),
        ],
        scratch_shapes=[pltpu.VMEM((2, 256), jnp.float32)],
    )(xyz1, xyz2, points1, points2, W1, b1[:, None])

    sc1, sh1 = _affine(st1, g1, be1, float(B * N))

    h2, st2 = pl.pallas_call(
        functools.partial(_p2_kernel, nj=nj),
        grid=(B, nj),
        in_specs=[
            pl.BlockSpec((1, 256, bn), lambda b, j: (b, 0, j)),
            pl.BlockSpec((128, 256), lambda b, j: (0, 0)),
            pl.BlockSpec((128, 1), lambda b, j: (0, 0)),
            pl.BlockSpec((256, 1), lambda b, j: (0, 0)),
            pl.BlockSpec((256, 1), lambda b, j: (0, 0)),
        ],
        out_specs=[
            pl.BlockSpec((1, 128, bn), lambda b, j: (b, 0, j)),
            pl.BlockSpec((2, 128), lambda b, j: (0, 0)),
        ],
        out_shape=[
            jax.ShapeDtypeStruct((B, 128, N), jnp.float32),
            jax.ShapeDtypeStruct((2, 128), jnp.float32),
        ],
        scratch_shapes=[pltpu.VMEM((2, 128), jnp.float32)],
    )(h1, W2, b2[:, None], sc1, sh1)

    sc2, sh2 = _affine(st2, g2, be2, float(B * N))

    out = pl.pallas_call(
        _p3_kernel,
        grid=(B, nj),
        in_specs=[
            pl.BlockSpec((1, 128, bn), lambda b, j: (b, 0, j)),
            pl.BlockSpec((128, 1), lambda b, j: (0, 0)),
            pl.BlockSpec((128, 1), lambda b, j: (0, 0)),
        ],
        out_specs=pl.BlockSpec((1, 128, bn), lambda b, j: (b, 0, j)),
        out_shape=jax.ShapeDtypeStruct((B, 128, N), jnp.float32),
    )(h2, sc2, sh2)

    return out


# speedup vs baseline: 19.4630x; 1.1709x over previous
import functools
import jax
import jax.numpy as jnp
from jax import lax
from jax.experimental import pallas as pl
from jax.experimental.pallas import tpu as pltpu

BN_BLK = 512
S = 1024
HI = 3.4e38


def _top3(dists, iota_s):
    d = dists
    vals, idxs = [], []
    for k in range(3):
        m = jnp.min(d, axis=0)
        is_min = d == m[None, :]
        ik = jnp.min(jnp.where(is_min, iota_s, S), axis=0)
        vals.append(m)
        idxs.append(ik)
        if k < 2:
            d = jnp.where(iota_s == ik[None, :], HI, d)
    return vals, idxs


def _p1_kernel(x1_ref, x2_ref, p1_ref, p2_ref, w1_ref, b1_ref, h1_ref,
               st1_ref, acc_ref, *, nj):
    b = pl.program_id(0)
    j = pl.program_id(1)

    x1 = x1_ref[0]
    x2 = x2_ref[0]
    dots = jax.lax.dot_general(x2, x1, (((0,), (0,)), ((), ())),
                               preferred_element_type=jnp.float32,
                               precision=lax.Precision.DEFAULT)
    x1sq = jnp.sum(x1 * x1, axis=0)
    x2sq = jnp.sum(x2 * x2, axis=0)
    dists = (-2.0 * dots + x1sq[None, :]) + x2sq[:, None]

    iota_s = lax.broadcasted_iota(jnp.int32, dists.shape, 0)
    vals, idxs = _top3(dists, iota_s)

    r0 = 1.0 / (vals[0] + 1e-8)
    r1 = 1.0 / (vals[1] + 1e-8)
    r2 = 1.0 / (vals[2] + 1e-8)
    norm = r0 + r1 + r2
    w0, w1w, w2w = r0 / norm, r1 / norm, r2 / norm

    onehot = (jnp.where(iota_s == idxs[0][None, :], w0[None, :], 0.0)
              + jnp.where(iota_s == idxs[1][None, :], w1w[None, :], 0.0)
              + jnp.where(iota_s == idxs[2][None, :], w2w[None, :], 0.0))

    p2 = p2_ref[0]
    interp = jax.lax.dot_general(p2, onehot, (((1,), (0,)), ((), ())),
                                 preferred_element_type=jnp.float32,
                                 precision=lax.Precision.HIGHEST)

    W1 = w1_ref[...]
    p1 = p1_ref[0]
    h1 = (jax.lax.dot_general(W1[:, :128], p1, (((1,), (0,)), ((), ())),
                              preferred_element_type=jnp.float32,
                              precision=lax.Precision.DEFAULT)
          + jax.lax.dot_general(W1[:, 128:], interp, (((1,), (0,)), ((), ())),
                                preferred_element_type=jnp.float32,
                                precision=lax.Precision.DEFAULT)
          + b1_ref[...])
    h1_ref[0] = h1

    @pl.when(jnp.logical_and(b == 0, j == 0))
    def _init():
        acc_ref[...] = jnp.zeros_like(acc_ref)

    acc_ref[0] += h1
    acc_ref[1] += h1 * h1

    @pl.when(jnp.logical_and(b == pl.num_programs(0) - 1, j == nj - 1))
    def _fin():
        st1_ref[0, :] = jnp.sum(acc_ref[0], axis=1)
        st1_ref[1, :] = jnp.sum(acc_ref[1], axis=1)


def _p2_kernel(h1_ref, w2_ref, b2_ref, sc1_ref, sh1_ref, h2_ref, st2_ref,
               acc_ref, *, nj):
    b = pl.program_id(0)
    j = pl.program_id(1)
    a1 = jnp.maximum(sc1_ref[...] * h1_ref[0] + sh1_ref[...], 0.0)
    h2 = (jax.lax.dot_general(w2_ref[...], a1, (((1,), (0,)), ((), ())),
                              preferred_element_type=jnp.float32,
                              precision=lax.Precision.DEFAULT)
          + b2_ref[...])
    h2_ref[0] = h2

    @pl.when(jnp.logical_and(b == 0, j == 0))
    def _init():
        acc_ref[...] = jnp.zeros_like(acc_ref)

    acc_ref[0] += h2
    acc_ref[1] += h2 * h2

    @pl.when(jnp.logical_and(b == pl.num_programs(0) - 1, j == nj - 1))
    def _fin():
        st2_ref[0, :] = jnp.sum(acc_ref[0], axis=1)
        st2_ref[1, :] = jnp.sum(acc_ref[1], axis=1)


def _p3_kernel(h2_ref, sc2_ref, sh2_ref, o_ref):
    o_ref[0] = jnp.maximum(sc2_ref[...] * h2_ref[0] + sh2_ref[...], 0.0)


def _affine(stats, g, be, count):
    mean = stats[0] / count
    var = stats[1] / count - mean * mean
    scale = g / jnp.sqrt(var + 1e-5)
    shift = be - scale * mean
    return scale[:, None], shift[:, None]


def kernel(xyz1, xyz2, points1, points2, W1, b1, g1, be1, W2, b2, g2, be2):
    B, _, N = xyz1.shape
    D2 = points2.shape[1]
    bn = BN_BLK
    nj = N // bn

    h1, st1 = pl.pallas_call(
        functools.partial(_p1_kernel, nj=nj),
        grid=(B, nj),
        in_specs=[
            pl.BlockSpec((1, 3, bn), lambda b, j: (b, 0, j)),
            pl.BlockSpec((1, 3, S), lambda b, j: (b, 0, 0)),
            pl.BlockSpec((1, 128, bn), lambda b, j: (b, 0, j)),
            pl.BlockSpec((1, D2, S), lambda b, j: (b, 0, 0)),
            pl.BlockSpec((256, 384), lambda b, j: (0, 0)),
            pl.BlockSpec((256, 1), lambda b, j: (0, 0)),
        ],
        out_specs=[
            pl.BlockSpec((1, 256, bn), lambda b, j: (b, 0, j)),
            pl.BlockSpec((2, 256), lambda b, j: (0, 0)),
        ],
        out_shape=[
            jax.ShapeDtypeStruct((B, 256, N), jnp.float32),
            jax.ShapeDtypeStruct((2, 256), jnp.float32),
        ],
        scratch_shapes=[pltpu.VMEM((2, 256, BN_BLK), jnp.float32)],
    )(xyz1, xyz2, points1, points2, W1, b1[:, None])

    sc1, sh1 = _affine(st1, g1, be1, float(B * N))

    h2, st2 = pl.pallas_call(
        functools.partial(_p2_kernel, nj=nj),
        grid=(B, nj),
        in_specs=[
            pl.BlockSpec((1, 256, bn), lambda b, j: (b, 0, j)),
            pl.BlockSpec((128, 256), lambda b, j: (0, 0)),
            pl.BlockSpec((128, 1), lambda b, j: (0, 0)),
            pl.BlockSpec((256, 1), lambda b, j: (0, 0)),
            pl.BlockSpec((256, 1), lambda b, j: (0, 0)),
        ],
        out_specs=[
            pl.BlockSpec((1, 128, bn), lambda b, j: (b, 0, j)),
            pl.BlockSpec((2, 128), lambda b, j: (0, 0)),
        ],
        out_shape=[
            jax.ShapeDtypeStruct((B, 128, N), jnp.float32),
            jax.ShapeDtypeStruct((2, 128), jnp.float32),
        ],
        scratch_shapes=[pltpu.VMEM((2, 128, BN_BLK), jnp.float32)],
    )(h1, W2, b2[:, None], sc1, sh1)

    sc2, sh2 = _affine(st2, g2, be2, float(B * N))

    out = pl.pallas_call(
        _p3_kernel,
        grid=(B, nj),
        in_specs=[
            pl.BlockSpec((1, 128, bn), lambda b, j: (b, 0, j)),
            pl.BlockSpec((128, 1), lambda b, j: (0, 0)),
            pl.BlockSpec((128, 1), lambda b, j: (0, 0)),
        ],
        out_specs=pl.BlockSpec((1, 128, bn), lambda b, j: (b, 0, j)),
        out_shape=jax.ShapeDtypeStruct((B, 128, N), jnp.float32),
    )(h2, sc2, sh2)

    return out
